# baseline (speedup 1.0000x reference)
import jax
import jax.numpy as jnp
from jax import lax
from jax.experimental import pallas as pl
from jax.experimental.pallas import tpu as pltpu
from jax.experimental.pallas import tpu_sc as plsc

_Q = 8000
_S = 2000
_N = 10000
_EMB = 128
_E = 320000

_AUG = 144
_DEG = _AUG - _EMB
_NSC = 2
_NTILE = 16
_NW = _NSC * _NTILE
_EPT = _E // _NW
_CHUNK = 40
_NCHUNK = _EPT // _CHUNK
_RPT = _Q // _NTILE
_SPT = _S // _NTILE
_CMB = _Q


def _segments(total):
    segs = [(off, _CHUNK) for off in range(0, total - total % _CHUNK, _CHUNK)]
    if total % _CHUNK:
        segs.append((total - total % _CHUNK, total % _CHUNK))
    return segs


def _sc_body(skl_hbm, edges_hbm, feat_hbm, deg_hbm,
             sidx, didx, rows_a, rows_b, gs_a, gs_b, ss_a, ss_b, comb_sh):
    cid = lax.axis_index("c")
    sid = lax.axis_index("s")
    w = cid * _NTILE + sid

    pltpu.sync_copy(edges_hbm.at[0, pl.ds(w * _EPT, _EPT)], sidx)
    pltpu.sync_copy(edges_hbm.at[1, pl.ds(w * _EPT, _EPT)], didx)

    def _rebase(i, carry):
        sl = pl.ds(i * 16, 16)
        didx[sl] = didx[sl] - _Q
        return carry

    lax.fori_loop(0, _EPT // 16, _rebase, 0)

    def _zrow(r, carry):
        for k in range(_AUG // 16):
            rows_a[r, pl.ds(k * 16, 16)] = jnp.zeros((16,), jnp.float32)
        return carry

    lax.fori_loop(0, _CHUNK, _zrow, 0)
    for off, sz in _segments(_RPT):
        pltpu.sync_copy(rows_a.at[pl.ds(0, sz)],
                        comb_sh.at[pl.ds(sid * _RPT + off, sz)])

    plsc.subcore_barrier()

    def _gather(c, buf, sem):
        pltpu.async_copy(skl_hbm.at[didx.at[pl.ds(c * _CHUNK, _CHUNK)]], buf, sem)

    def _gather_wait(c, buf, sem):
        pltpu.make_async_copy(skl_hbm.at[didx.at[pl.ds(c * _CHUNK, _CHUNK)]],
                              buf, sem).wait()

    def _scatter(c, buf, sem):
        pltpu.async_copy(buf, comb_sh.at[sidx.at[pl.ds(c * _CHUNK, _CHUNK)]],
                         sem, add=True)

    def _scatter_wait(c, buf, sem):
        pltpu.make_async_copy(buf,
                              comb_sh.at[sidx.at[pl.ds(c * _CHUNK, _CHUNK)]],
                              sem).wait()

    _gather(0, rows_a, gs_a)
    _gather(1, rows_b, gs_b)
    _gather_wait(0, rows_a, gs_a)
    _scatter(0, rows_a, ss_a)
    _gather_wait(1, rows_b, gs_b)
    _scatter(1, rows_b, ss_b)

    def _pipe(i, carry):
        c = 2 + 2 * i
        _scatter_wait(c - 2, rows_a, ss_a)
        _gather(c, rows_a, gs_a)
        _scatter_wait(c - 1, rows_b, ss_b)
        _gather(c + 1, rows_b, gs_b)
        _gather_wait(c, rows_a, gs_a)
        _scatter(c, rows_a, ss_a)
        _gather_wait(c + 1, rows_b, gs_b)
        _scatter(c + 1, rows_b, ss_b)
        return carry

    lax.fori_loop(0, (_NCHUNK - 2) // 2, _pipe, 0)
    _scatter_wait(_NCHUNK - 2, rows_a, ss_a)
    _scatter_wait(_NCHUNK - 1, rows_b, ss_b)
    plsc.subcore_barrier()

    for off, sz in _segments(_RPT):
        base = sid * _RPT + off
        pltpu.sync_copy(comb_sh.at[pl.ds(base, sz)], rows_a.at[pl.ds(0, sz)])
        pltpu.sync_copy(rows_a.at[pl.ds(0, sz), pl.ds(0, _EMB)],
                        feat_hbm.at[cid, pl.ds(base, sz)])
        pltpu.sync_copy(rows_a.at[pl.ds(0, sz), pl.ds(_EMB, _DEG)],
                        deg_hbm.at[cid, pl.ds(base, sz)])


_sc_aggregate = pl.kernel(
    _sc_body,
    out_type=(jax.ShapeDtypeStruct((_NSC, _Q, _EMB), jnp.float32),
              jax.ShapeDtypeStruct((_NSC, _Q, _DEG), jnp.float32)),
    mesh=plsc.VectorSubcoreMesh(core_axis_name="c", subcore_axis_name="s",
                                num_cores=_NSC),
    scratch_types=[
        pltpu.VMEM((_EPT,), jnp.int32),
        pltpu.VMEM((_EPT,), jnp.int32),
        pltpu.VMEM((_CHUNK, _AUG), jnp.float32),
        pltpu.VMEM((_CHUNK, _AUG), jnp.float32),
        pltpu.SemaphoreType.DMA,
        pltpu.SemaphoreType.DMA,
        pltpu.SemaphoreType.DMA,
        pltpu.SemaphoreType.DMA,
        pltpu.VMEM_SHARED((_CMB, _AUG), jnp.float32),
    ],
    compiler_params=pltpu.CompilerParams(use_tc_tiling_on_sc=False),
)

_BLK = 1000
_NB = _Q // _BLK


def _head_body(feat_ref, deg_ref, nodes_ref, w1_ref, w2_ref, bias_ref, out_ref):
    j = pl.program_id(0)

    @pl.when(j < 2 * _NB)
    def _compute():
        agg = feat_ref[0]
        deg = deg_ref[0, :, 0:1]
        for p in range(1, _NSC):
            agg = agg + feat_ref[p]
            deg = deg + deg_ref[p, :, 0:1]
        aggn = agg / jnp.maximum(deg, 1.0)
        h = jnp.dot(aggn, w1_ref[...], preferred_element_type=jnp.float32)
        h = h + jnp.dot(nodes_ref[...], w2_ref[...],
                        preferred_element_type=jnp.float32)
        h = jnp.maximum(h, 0.0)
        out_ref[...] = h + bias_ref[0:1, :]

    @pl.when(j == 2 * _NB)
    def _pad_row():
        out_ref[...] = jnp.zeros((_BLK, _EMB), jnp.float32)


_head = pl.pallas_call(
    _head_body,
    grid=(2 * _NB + 1,),
    in_specs=[
        pl.BlockSpec((_NSC, _BLK, _EMB), lambda j: (0, j % _NB, 0)),
        pl.BlockSpec((_NSC, _BLK, _DEG), lambda j: (0, j % _NB, 0)),
        pl.BlockSpec((_BLK, _EMB), lambda j: (j % _NB, 0)),
        pl.BlockSpec((_EMB, _EMB), lambda j: (0, 0)),
        pl.BlockSpec((_EMB, _EMB), lambda j: (0, 0)),
        pl.BlockSpec((8, _EMB), lambda j: (j // _NB, 0)),
    ],
    out_specs=pl.BlockSpec((_BLK, _EMB), lambda j: (j, 0)),
    out_shape=jax.ShapeDtypeStruct((2 * _Q + 1, _EMB), jnp.float32),
)


def kernel(nodes_features, edge_index, W1, W2, correct_bias, incorrect_bias):
    nf = nodes_features.astype(jnp.float32)
    edges = edge_index.astype(jnp.int32)

    skl_aug = jnp.concatenate(
        [nf[_Q:],
         jnp.ones((_S, 1), jnp.float32),
         jnp.zeros((_S, _AUG - _EMB - 1), jnp.float32)], axis=1)

    feat, deg = _sc_aggregate(skl_aug, edges)

    bias3 = jnp.concatenate(
        [incorrect_bias.astype(jnp.float32),
         jnp.zeros((7, _EMB), jnp.float32),
         correct_bias.astype(jnp.float32),
         jnp.zeros((15, _EMB), jnp.float32)], axis=0)

    return _head(feat, deg, nf, W1.astype(jnp.float32),
                 W2.astype(jnp.float32), bias3)


# speedup vs baseline: 1.2160x; 1.2160x over previous
import jax
import jax.numpy as jnp
from jax import lax
from jax.experimental import pallas as pl
from jax.experimental.pallas import tpu as pltpu
from jax.experimental.pallas import tpu_sc as plsc

_Q = 8000
_S = 2000
_N = 10000
_EMB = 128
_E = 320000

_AUG = 144
_DEG = _AUG - _EMB
_NSC = 2
_NTILE = 16
_NW = _NSC * _NTILE
_EPT = _E // _NW
_CHUNK = 64
_NCHUNK = _EPT // _CHUNK
_TAIL = _EPT - _NCHUNK * _CHUNK
_RPT = _Q // _NTILE
_SPT = _S // _NTILE
_CMB = _Q + _S


def _segments(total):
    segs = [(off, _CHUNK) for off in range(0, total - total % _CHUNK, _CHUNK)]
    if total % _CHUNK:
        segs.append((total - total % _CHUNK, total % _CHUNK))
    return segs


def _sc_body(skl_hbm, edges_hbm, feat_hbm, deg_hbm,
             sidx, didx, rows_a, rows_b, gs_a, gs_b, ss_a, ss_b, comb_sh):
    cid = lax.axis_index("c")
    sid = lax.axis_index("s")
    w = cid * _NTILE + sid

    pltpu.sync_copy(edges_hbm.at[0, pl.ds(w * _EPT, _EPT)], sidx)
    pltpu.sync_copy(edges_hbm.at[1, pl.ds(w * _EPT, _EPT)], didx)

    def _zrow(r, carry):
        for k in range(_AUG // 16):
            rows_a[r, pl.ds(k * 16, 16)] = jnp.zeros((16,), jnp.float32)
        return carry

    lax.fori_loop(0, _CHUNK, _zrow, 0)
    for off, sz in _segments(_RPT):
        pltpu.sync_copy(rows_a.at[pl.ds(0, sz)],
                        comb_sh.at[pl.ds(sid * _RPT + off, sz)])

    for off, sz in _segments(_SPT):
        pltpu.sync_copy(skl_hbm.at[pl.ds(sid * _SPT + off, sz)],
                        rows_a.at[pl.ds(0, sz)])
        pltpu.sync_copy(rows_a.at[pl.ds(0, sz)],
                        comb_sh.at[pl.ds(_Q + sid * _SPT + off, sz)])
    plsc.subcore_barrier()

    def _gather(c, buf, sem):
        pltpu.async_copy(comb_sh.at[didx.at[pl.ds(c * _CHUNK, _CHUNK)]], buf, sem)

    def _gather_wait(c, buf, sem):
        pltpu.make_async_copy(comb_sh.at[didx.at[pl.ds(c * _CHUNK, _CHUNK)]],
                              buf, sem).wait()

    def _scatter(c, buf, sem):
        pltpu.async_copy(buf, comb_sh.at[sidx.at[pl.ds(c * _CHUNK, _CHUNK)]],
                         sem, add=True)

    def _scatter_wait(c, buf, sem):
        pltpu.make_async_copy(buf,
                              comb_sh.at[sidx.at[pl.ds(c * _CHUNK, _CHUNK)]],
                              sem).wait()

    _gather(0, rows_a, gs_a)
    _gather(1, rows_b, gs_b)
    _gather_wait(0, rows_a, gs_a)
    _scatter(0, rows_a, ss_a)
    _gather_wait(1, rows_b, gs_b)
    _scatter(1, rows_b, ss_b)

    def _pipe(i, carry):
        c = 2 + 2 * i
        _scatter_wait(c - 2, rows_a, ss_a)
        _gather(c, rows_a, gs_a)
        _scatter_wait(c - 1, rows_b, ss_b)
        _gather(c + 1, rows_b, gs_b)
        _gather_wait(c, rows_a, gs_a)
        _scatter(c, rows_a, ss_a)
        _gather_wait(c + 1, rows_b, gs_b)
        _scatter(c + 1, rows_b, ss_b)
        return carry

    lax.fori_loop(0, (_NCHUNK - 2) // 2, _pipe, 0)
    _scatter_wait(_NCHUNK - 2, rows_a, ss_a)
    _scatter_wait(_NCHUNK - 1, rows_b, ss_b)
    if _TAIL:
        tl = pl.ds(_NCHUNK * _CHUNK, _TAIL)
        buf = rows_a.at[pl.ds(0, _TAIL)]
        pltpu.async_copy(comb_sh.at[didx.at[tl]], buf, gs_a).wait()
        pltpu.sync_copy(buf, comb_sh.at[sidx.at[tl]], add=True)
    plsc.subcore_barrier()

    for off, sz in _segments(_RPT):
        base = sid * _RPT + off
        pltpu.sync_copy(comb_sh.at[pl.ds(base, sz)], rows_a.at[pl.ds(0, sz)])
        pltpu.sync_copy(rows_a.at[pl.ds(0, sz), pl.ds(0, _EMB)],
                        feat_hbm.at[cid, pl.ds(base, sz)])
        pltpu.sync_copy(rows_a.at[pl.ds(0, sz), pl.ds(_EMB, _DEG)],
                        deg_hbm.at[cid, pl.ds(base, sz)])


_sc_aggregate = pl.kernel(
    _sc_body,
    out_type=(jax.ShapeDtypeStruct((_NSC, _Q, _EMB), jnp.float32),
              jax.ShapeDtypeStruct((_NSC, _Q, _DEG), jnp.float32)),
    mesh=plsc.VectorSubcoreMesh(core_axis_name="c", subcore_axis_name="s",
                                num_cores=_NSC),
    scratch_types=[
        pltpu.VMEM((_EPT,), jnp.int32),
        pltpu.VMEM((_EPT,), jnp.int32),
        pltpu.VMEM((_CHUNK, _AUG), jnp.float32),
        pltpu.VMEM((_CHUNK, _AUG), jnp.float32),
        pltpu.SemaphoreType.DMA,
        pltpu.SemaphoreType.DMA,
        pltpu.SemaphoreType.DMA,
        pltpu.SemaphoreType.DMA,
        pltpu.VMEM_SHARED((_CMB, _AUG), jnp.float32),
    ],
    compiler_params=pltpu.CompilerParams(use_tc_tiling_on_sc=False),
)

_BLK = 2000
_NB = _Q // _BLK


def _head_body(feat_ref, deg_ref, nodes_ref, w1_ref, w2_ref, bias_ref, out_ref):
    j = pl.program_id(0)

    @pl.when(j < 2 * _NB)
    def _compute():
        agg = feat_ref[0]
        deg = deg_ref[0, :, 0:1]
        for p in range(1, _NSC):
            agg = agg + feat_ref[p]
            deg = deg + deg_ref[p, :, 0:1]
        aggn = agg / jnp.maximum(deg, 1.0)
        h = jnp.dot(aggn, w1_ref[...], preferred_element_type=jnp.float32)
        h = h + jnp.dot(nodes_ref[...], w2_ref[...],
                        preferred_element_type=jnp.float32)
        h = jnp.maximum(h, 0.0)
        out_ref[...] = h + bias_ref[0:1, :]

    @pl.when(j == 2 * _NB)
    def _pad_row():
        out_ref[...] = jnp.zeros((_BLK, _EMB), jnp.float32)


_head = pl.pallas_call(
    _head_body,
    grid=(2 * _NB + 1,),
    in_specs=[
        pl.BlockSpec((_NSC, _BLK, _EMB), lambda j: (0, j % _NB, 0)),
        pl.BlockSpec((_NSC, _BLK, _DEG), lambda j: (0, j % _NB, 0)),
        pl.BlockSpec((_BLK, _EMB), lambda j: (j % _NB, 0)),
        pl.BlockSpec((_EMB, _EMB), lambda j: (0, 0)),
        pl.BlockSpec((_EMB, _EMB), lambda j: (0, 0)),
        pl.BlockSpec((8, _EMB), lambda j: (j // _NB, 0)),
    ],
    out_specs=pl.BlockSpec((_BLK, _EMB), lambda j: (j, 0)),
    out_shape=jax.ShapeDtypeStruct((2 * _Q + 1, _EMB), jnp.float32),
)


def kernel(nodes_features, edge_index, W1, W2, correct_bias, incorrect_bias):
    nf = nodes_features.astype(jnp.float32)
    edges = edge_index.astype(jnp.int32)

    skl_aug = jnp.concatenate(
        [nf[_Q:],
         jnp.ones((_S, 1), jnp.float32),
         jnp.zeros((_S, _AUG - _EMB - 1), jnp.float32)], axis=1)

    feat, deg = _sc_aggregate(skl_aug, edges)

    bias3 = jnp.concatenate(
        [incorrect_bias.astype(jnp.float32),
         jnp.zeros((7, _EMB), jnp.float32),
         correct_bias.astype(jnp.float32),
         jnp.zeros((15, _EMB), jnp.float32)], axis=0)

    return _head(feat, deg, nf, W1.astype(jnp.float32),
                 W2.astype(jnp.float32), bias3)


# speedup vs baseline: 1.2221x; 1.0050x over previous
import jax
import jax.numpy as jnp
from jax import lax
from jax.experimental import pallas as pl
from jax.experimental.pallas import tpu as pltpu
from jax.experimental.pallas import tpu_sc as plsc

_Q = 8000
_S = 2000
_N = 10000
_EMB = 128
_E = 320000

_AUG = 144
_DEG = _AUG - _EMB
_NSC = 2
_NTILE = 16
_NW = _NSC * _NTILE
_EPT = _E // _NW
_CHUNK = 64
_NCHUNK = _EPT // _CHUNK
_TAIL = _EPT - _NCHUNK * _CHUNK
_RPT = _Q // _NTILE
_SPT = _S // _NTILE
_CMB = _Q + _S


def _segments(total):
    segs = [(off, _CHUNK) for off in range(0, total - total % _CHUNK, _CHUNK)]
    if total % _CHUNK:
        segs.append((total - total % _CHUNK, total % _CHUNK))
    return segs


def _sc_body(skl_hbm, edges_hbm, feat_hbm, deg_hbm,
             sidx, didx, rows_a, rows_b, gs_a, gs_b, ss_a, ss_b, comb_sh):
    cid = lax.axis_index("c")
    sid = lax.axis_index("s")
    w = cid * _NTILE + sid

    pltpu.async_copy(edges_hbm.at[0, pl.ds(w * _EPT, _EPT)], sidx, gs_a)
    pltpu.async_copy(edges_hbm.at[1, pl.ds(w * _EPT, _EPT)], didx, gs_b)

    def _zrow(r, carry):
        for k in range(_AUG // 16):
            rows_a[r, pl.ds(k * 16, 16)] = jnp.zeros((16,), jnp.float32)
        return carry

    lax.fori_loop(0, _CHUNK, _zrow, 0)
    for off, sz in _segments(_RPT):
        pltpu.sync_copy(rows_a.at[pl.ds(0, sz)],
                        comb_sh.at[pl.ds(sid * _RPT + off, sz)])

    for off, sz in _segments(_SPT):
        pltpu.sync_copy(skl_hbm.at[pl.ds(sid * _SPT + off, sz)],
                        rows_a.at[pl.ds(0, sz)])
        pltpu.sync_copy(rows_a.at[pl.ds(0, sz)],
                        comb_sh.at[pl.ds(_Q + sid * _SPT + off, sz)])
    pltpu.make_async_copy(edges_hbm.at[0, pl.ds(w * _EPT, _EPT)], sidx,
                          gs_a).wait()
    pltpu.make_async_copy(edges_hbm.at[1, pl.ds(w * _EPT, _EPT)], didx,
                          gs_b).wait()
    plsc.subcore_barrier()

    def _gather(c, buf, sem):
        pltpu.async_copy(comb_sh.at[didx.at[pl.ds(c * _CHUNK, _CHUNK)]], buf, sem)

    def _gather_wait(c, buf, sem):
        pltpu.make_async_copy(comb_sh.at[didx.at[pl.ds(c * _CHUNK, _CHUNK)]],
                              buf, sem).wait()

    def _scatter(c, buf, sem):
        pltpu.async_copy(buf, comb_sh.at[sidx.at[pl.ds(c * _CHUNK, _CHUNK)]],
                         sem, add=True)

    def _scatter_wait(c, buf, sem):
        pltpu.make_async_copy(buf,
                              comb_sh.at[sidx.at[pl.ds(c * _CHUNK, _CHUNK)]],
                              sem).wait()

    _gather(0, rows_a, gs_a)
    _gather(1, rows_b, gs_b)
    _gather_wait(0, rows_a, gs_a)
    _scatter(0, rows_a, ss_a)
    _gather_wait(1, rows_b, gs_b)
    _scatter(1, rows_b, ss_b)

    def _pipe(i, carry):
        c = 2 + 2 * i
        _scatter_wait(c - 2, rows_a, ss_a)
        _gather(c, rows_a, gs_a)
        _scatter_wait(c - 1, rows_b, ss_b)
        _gather(c + 1, rows_b, gs_b)
        _gather_wait(c, rows_a, gs_a)
        _scatter(c, rows_a, ss_a)
        _gather_wait(c + 1, rows_b, gs_b)
        _scatter(c + 1, rows_b, ss_b)
        return carry

    lax.fori_loop(0, (_NCHUNK - 2) // 2, _pipe, 0)
    _scatter_wait(_NCHUNK - 2, rows_a, ss_a)
    _scatter_wait(_NCHUNK - 1, rows_b, ss_b)
    if _TAIL:
        tl = pl.ds(_NCHUNK * _CHUNK, _TAIL)
        buf = rows_a.at[pl.ds(0, _TAIL)]
        pltpu.async_copy(comb_sh.at[didx.at[tl]], buf, gs_a).wait()
        pltpu.sync_copy(buf, comb_sh.at[sidx.at[tl]], add=True)
    plsc.subcore_barrier()

    for off, sz in _segments(_RPT):
        base = sid * _RPT + off
        pltpu.sync_copy(comb_sh.at[pl.ds(base, sz)], rows_a.at[pl.ds(0, sz)])
        pltpu.sync_copy(rows_a.at[pl.ds(0, sz), pl.ds(0, _EMB)],
                        feat_hbm.at[cid, pl.ds(base, sz)])
        pltpu.sync_copy(rows_a.at[pl.ds(0, sz), pl.ds(_EMB, _DEG)],
                        deg_hbm.at[cid, pl.ds(base, sz)])


_sc_aggregate = pl.kernel(
    _sc_body,
    out_type=(jax.ShapeDtypeStruct((_NSC, _Q, _EMB), jnp.float32),
              jax.ShapeDtypeStruct((_NSC, _Q, _DEG), jnp.float32)),
    mesh=plsc.VectorSubcoreMesh(core_axis_name="c", subcore_axis_name="s",
                                num_cores=_NSC),
    scratch_types=[
        pltpu.VMEM((_EPT,), jnp.int32),
        pltpu.VMEM((_EPT,), jnp.int32),
        pltpu.VMEM((_CHUNK, _AUG), jnp.float32),
        pltpu.VMEM((_CHUNK, _AUG), jnp.float32),
        pltpu.SemaphoreType.DMA,
        pltpu.SemaphoreType.DMA,
        pltpu.SemaphoreType.DMA,
        pltpu.SemaphoreType.DMA,
        pltpu.VMEM_SHARED((_CMB, _AUG), jnp.float32),
    ],
    compiler_params=pltpu.CompilerParams(use_tc_tiling_on_sc=False),
)

_BLK = 4000
_NB = _Q // _BLK


def _head_body(feat_ref, deg_ref, nodes_ref, w1_ref, w2_ref, bias_ref, out_ref):
    j = pl.program_id(0)

    @pl.when(j < 2 * _NB)
    def _compute():
        agg = feat_ref[0]
        deg = deg_ref[0, :, 0:1]
        for p in range(1, _NSC):
            agg = agg + feat_ref[p]
            deg = deg + deg_ref[p, :, 0:1]
        aggn = agg / jnp.maximum(deg, 1.0)
        h = jnp.dot(aggn, w1_ref[...], preferred_element_type=jnp.float32)
        h = h + jnp.dot(nodes_ref[...], w2_ref[...],
                        preferred_element_type=jnp.float32)
        h = jnp.maximum(h, 0.0)
        out_ref[...] = h + bias_ref[0:1, :]

    @pl.when(j == 2 * _NB)
    def _pad_row():
        out_ref[...] = jnp.zeros((_BLK, _EMB), jnp.float32)


_head = pl.pallas_call(
    _head_body,
    grid=(2 * _NB + 1,),
    in_specs=[
        pl.BlockSpec((_NSC, _BLK, _EMB), lambda j: (0, j % _NB, 0)),
        pl.BlockSpec((_NSC, _BLK, _DEG), lambda j: (0, j % _NB, 0)),
        pl.BlockSpec((_BLK, _EMB), lambda j: (j % _NB, 0)),
        pl.BlockSpec((_EMB, _EMB), lambda j: (0, 0)),
        pl.BlockSpec((_EMB, _EMB), lambda j: (0, 0)),
        pl.BlockSpec((8, _EMB), lambda j: (j // _NB, 0)),
    ],
    out_specs=pl.BlockSpec((_BLK, _EMB), lambda j: (j, 0)),
    out_shape=jax.ShapeDtypeStruct((2 * _Q + 1, _EMB), jnp.float32),
)


def kernel(nodes_features, edge_index, W1, W2, correct_bias, incorrect_bias):
    nf = nodes_features.astype(jnp.float32)
    edges = edge_index.astype(jnp.int32)

    skl_aug = jnp.concatenate(
        [nf[_Q:],
         jnp.ones((_S, 1), jnp.float32),
         jnp.zeros((_S, _AUG - _EMB - 1), jnp.float32)], axis=1)

    feat, deg = _sc_aggregate(skl_aug, edges)

    bias3 = jnp.concatenate(
        [incorrect_bias.astype(jnp.float32),
         jnp.zeros((7, _EMB), jnp.float32),
         correct_bias.astype(jnp.float32),
         jnp.zeros((15, _EMB), jnp.float32)], axis=0)

    return _head(feat, deg, nf, W1.astype(jnp.float32),
                 W2.astype(jnp.float32), bias3)
